# Initial kernel scaffold; baseline (speedup 1.0000x reference)
import jax
import jax.numpy as jnp
from jax.experimental import pallas as pl


def kernel(x, edge_index, W_l1, b_l1, W_r1, W_res1, b_res1, W_l2, b_l2, W_r2, W_res2, b_res2):
    raise NotImplementedError("write your pallas kernel here")


# speedup vs baseline: 2.9053x; 2.9053x over previous
import jax
import jax.numpy as jnp
from jax import lax
from jax.experimental import pallas as pl
from jax.experimental.pallas import tpu as pltpu
from jax.experimental.pallas import tpu_sc as plsc

N = 10000
E = 160000
D = 256
DH = 128
RW = 0.001

NC = 2
NS = 16
HALF = N // NC
ROWS_PER_TILE = 320
PAD = NS * ROWS_PER_TILE
TRASH = HALF
EPT = E // NS
G = 80
STEPS = EPT // G


def _sc_body(xc_hbm, src_hbm, dst_hbm, out_hbm, src_v, dst_v, rows_v,
             stage_v, sem, agg_sh):
  c = lax.axis_index("c")
  s = lax.axis_index("s")

  zero16 = jnp.zeros((16,), jnp.float32)

  def zrow(i, _):
    def zcol(j, _):
      stage_v[i, pl.ds(j * 16, 16)] = zero16
      return 0
    return lax.fori_loop(0, DH // 16, zcol, 0)

  lax.fori_loop(0, ROWS_PER_TILE, zrow, 0)

  pltpu.sync_copy(src_hbm.at[s], src_v)
  pltpu.sync_copy(dst_hbm.at[c, s], dst_v)

  for p in range(2):
    pltpu.sync_copy(stage_v,
                    agg_sh.at[pl.ds(s * ROWS_PER_TILE, ROWS_PER_TILE)])
    plsc.subcore_barrier()

    def step(j, _):
      pltpu.async_copy(xc_hbm.at[p].at[src_v.at[j]], rows_v, sem).wait()
      pltpu.sync_copy(rows_v, agg_sh.at[dst_v.at[j]], add=True)
      return 0

    lax.fori_loop(0, STEPS, step, 0)
    plsc.subcore_barrier()

    pltpu.sync_copy(agg_sh.at[pl.ds(s * ROWS_PER_TILE, ROWS_PER_TILE)],
                    stage_v)
    pltpu.sync_copy(stage_v,
                    out_hbm.at[c, p, pl.ds(s * ROWS_PER_TILE, ROWS_PER_TILE)])
    if p == 0:
      lax.fori_loop(0, ROWS_PER_TILE, zrow, 0)


@jax.jit
def _segment_sum_sc(xc, src3, dst4):
  mesh = plsc.VectorSubcoreMesh(core_axis_name="c", subcore_axis_name="s",
                                num_cores=NC, num_subcores=NS)
  f = pl.kernel(
      _sc_body,
      out_type=jax.ShapeDtypeStruct((NC, 2, PAD, DH), jnp.float32),
      mesh=mesh,
      scratch_types=[
          pltpu.VMEM((STEPS, G), jnp.int32),
          pltpu.VMEM((STEPS, G), jnp.int32),
          pltpu.VMEM((G, DH), jnp.float32),
          pltpu.VMEM((ROWS_PER_TILE, DH), jnp.float32),
          pltpu.SemaphoreType.DMA,
          pltpu.VMEM_SHARED((PAD, DH), jnp.float32),
      ],
  )
  return f(xc, src3, dst4)


def _tc_body(x_ref, agg_ref, wl1, wr1, wres1, wl2, wr2, wres2,
             bl1, bres1, bl2, bres2, loc_ref, scale_ref):
  a1 = RW * wl1[...]
  b1 = RW * wr1[...] + (1.0 - RW) * wres1[...]
  a2 = RW * wl2[...]
  b2 = RW * wr2[...] + (1.0 - RW) * wres2[...]
  c1 = RW * bl1[...] + (1.0 - RW) * bres1[...]
  c2 = RW * bl2[...] + (1.0 - RW) * bres2[...]
  hs = jnp.concatenate([agg_ref[...], x_ref[...]], axis=1)
  wcat = jnp.concatenate(
      [jnp.concatenate([a1, b1], axis=1),
       jnp.concatenate([a2, b2], axis=1)], axis=0)
  hall = lax.dot_general(hs, wcat, (((1,), (1,)), ((), ())),
                         preferred_element_type=jnp.float32)
  h1 = hall[:, :D] + c1
  h2 = hall[:, D:] + c2
  loc_ref[...] = jnp.clip(h1, -100.0, 100.0)
  scale_ref[...] = jnp.minimum(jax.nn.softplus(h2) + 0.001, 100.0)


@jax.jit
def _dense_tc(x, agg, W_l1, W_r1, W_res1, W_l2, W_r2, W_res2,
              b_l1, b_res1, b_l2, b_res2):
  bm = 1000
  grid = (N // bm,)
  row = pl.BlockSpec((bm, D), lambda i: (i, 0))
  full = pl.BlockSpec((D, D), lambda i: (0, 0))
  vec = pl.BlockSpec((1, D), lambda i: (0, 0))
  return pl.pallas_call(
      _tc_body,
      grid=grid,
      in_specs=[row, row, full, full, full, full, full, full,
                vec, vec, vec, vec],
      out_specs=[row, row],
      out_shape=[jax.ShapeDtypeStruct((N, D), jnp.float32),
                 jax.ShapeDtypeStruct((N, D), jnp.float32)],
  )(x, agg, W_l1, W_r1, W_res1, W_l2, W_r2, W_res2,
    b_l1.reshape(1, D), b_res1.reshape(1, D),
    b_l2.reshape(1, D), b_res2.reshape(1, D))


def kernel(x, edge_index, W_l1, b_l1, W_r1, W_res1, b_res1,
           W_l2, b_l2, W_r2, W_res2, b_res2):
  src = edge_index[0]
  dst = edge_index[1]
  src3 = src.reshape(NS, STEPS, G)
  dst_c0 = jnp.where(dst < HALF, dst, TRASH)
  dst_c1 = jnp.where(dst >= HALF, dst - HALF, TRASH)
  dst4 = jnp.stack([dst_c0, dst_c1]).reshape(NC, NS, STEPS, G)
  xc = jnp.stack([x[:, :DH], x[:, DH:]])

  agg_pad = _segment_sum_sc(xc, src3, dst4)
  agg = jnp.concatenate(
      [jnp.concatenate([agg_pad[0, 0, :HALF], agg_pad[0, 1, :HALF]], axis=1),
       jnp.concatenate([agg_pad[1, 0, :HALF], agg_pad[1, 1, :HALF]], axis=1)],
      axis=0)

  loc, scale = _dense_tc(x, agg, W_l1, W_r1, W_res1, W_l2, W_r2, W_res2,
                         b_l1, b_res1, b_l2, b_res2)
  return (loc, scale)


# speedup vs baseline: 4.1756x; 1.4372x over previous
import jax
import jax.numpy as jnp
from jax import lax
from jax.experimental import pallas as pl
from jax.experimental.pallas import tpu as pltpu
from jax.experimental.pallas import tpu_sc as plsc

N = 10000
E = 160000
D = 256
DH = 128
RW = 0.001

NC = 2
NS = 16
HALF = N // NC
ROWS_PER_TILE = 320
PAD = NS * ROWS_PER_TILE
TRASH = HALF
EPT = E // NS
G = 80
STEPS = EPT // G
SB = 64


def _sc_body(xc_hbm, src_hbm, dst_hbm, out_hbm, src_v, dst_v, rows_v,
             stage_v, sem0, sem1, agg_sh):
  c = lax.axis_index("c")
  s = lax.axis_index("s")

  zero16 = jnp.zeros((16,), jnp.float32)

  def zrow(i, _):
    def zcol(j, _):
      stage_v[i, pl.ds(j * 16, 16)] = zero16
      return 0
    return lax.fori_loop(0, DH // 16, zcol, 0)

  lax.fori_loop(0, SB, zrow, 0)

  pltpu.sync_copy(src_hbm.at[s], src_v)
  pltpu.sync_copy(dst_hbm.at[c, s], dst_v)

  for p in range(2):
    def zs(q, _):
      pltpu.sync_copy(stage_v,
                      agg_sh.at[pl.ds(s * ROWS_PER_TILE + q * SB, SB)])
      return 0

    lax.fori_loop(0, ROWS_PER_TILE // SB, zs, 0)
    plsc.subcore_barrier()

    pltpu.async_copy(xc_hbm.at[p].at[src_v.at[0]], rows_v.at[0], sem0)

    def step(j, _):
      even = lax.rem(j, 2) == 0

      @pl.when((j + 1 < STEPS) & even)
      def _():
        pltpu.async_copy(xc_hbm.at[p].at[src_v.at[j + 1]], rows_v.at[1],
                         sem1)

      @pl.when((j + 1 < STEPS) & jnp.logical_not(even))
      def _():
        pltpu.async_copy(xc_hbm.at[p].at[src_v.at[j + 1]], rows_v.at[0],
                         sem0)

      @pl.when(even)
      def _():
        pltpu.make_async_copy(xc_hbm.at[p].at[src_v.at[j]], rows_v.at[0],
                              sem0).wait()
        pltpu.sync_copy(rows_v.at[0], agg_sh.at[dst_v.at[j]], add=True)

      @pl.when(jnp.logical_not(even))
      def _():
        pltpu.make_async_copy(xc_hbm.at[p].at[src_v.at[j]], rows_v.at[1],
                              sem1).wait()
        pltpu.sync_copy(rows_v.at[1], agg_sh.at[dst_v.at[j]], add=True)

      return 0

    lax.fori_loop(0, STEPS, step, 0)
    plsc.subcore_barrier()

    def co(q, _):
      pltpu.sync_copy(agg_sh.at[pl.ds(s * ROWS_PER_TILE + q * SB, SB)],
                      stage_v)
      pltpu.sync_copy(stage_v,
                      out_hbm.at[c, p, pl.ds(s * ROWS_PER_TILE + q * SB, SB)])
      return 0

    lax.fori_loop(0, ROWS_PER_TILE // SB, co, 0)
    if p == 0:
      lax.fori_loop(0, SB, zrow, 0)


@jax.jit
def _segment_sum_sc(xc, src3, dst4):
  mesh = plsc.VectorSubcoreMesh(core_axis_name="c", subcore_axis_name="s",
                                num_cores=NC, num_subcores=NS)
  f = pl.kernel(
      _sc_body,
      out_type=jax.ShapeDtypeStruct((NC, 2, PAD, DH), jnp.float32),
      mesh=mesh,
      scratch_types=[
          pltpu.VMEM((STEPS, G), jnp.int32),
          pltpu.VMEM((STEPS, G), jnp.int32),
          pltpu.VMEM((2, G, DH), jnp.float32),
          pltpu.VMEM((SB, DH), jnp.float32),
          pltpu.SemaphoreType.DMA,
          pltpu.SemaphoreType.DMA,
          pltpu.VMEM_SHARED((PAD, DH), jnp.float32),
      ],
  )
  return f(xc, src3, dst4)


def _tc_body(x_ref, agg_ref, wl1, wr1, wres1, wl2, wr2, wres2,
             bl1, bres1, bl2, bres2, loc_ref, scale_ref):
  a1 = RW * wl1[...]
  b1 = RW * wr1[...] + (1.0 - RW) * wres1[...]
  a2 = RW * wl2[...]
  b2 = RW * wr2[...] + (1.0 - RW) * wres2[...]
  c1 = RW * bl1[...] + (1.0 - RW) * bres1[...]
  c2 = RW * bl2[...] + (1.0 - RW) * bres2[...]
  hs = jnp.concatenate([agg_ref[...], x_ref[...]], axis=1)
  wcat = jnp.concatenate(
      [jnp.concatenate([a1, b1], axis=1),
       jnp.concatenate([a2, b2], axis=1)], axis=0)
  hall = lax.dot_general(hs, wcat, (((1,), (1,)), ((), ())),
                         preferred_element_type=jnp.float32)
  h1 = hall[:, :D] + c1
  h2 = hall[:, D:] + c2
  loc_ref[...] = jnp.clip(h1, -100.0, 100.0)
  scale_ref[...] = jnp.minimum(jax.nn.softplus(h2) + 0.001, 100.0)


@jax.jit
def _dense_tc(x, agg, W_l1, W_r1, W_res1, W_l2, W_r2, W_res2,
              b_l1, b_res1, b_l2, b_res2):
  bm = 1000
  grid = (N // bm,)
  row = pl.BlockSpec((bm, D), lambda i: (i, 0))
  full = pl.BlockSpec((D, D), lambda i: (0, 0))
  vec = pl.BlockSpec((1, D), lambda i: (0, 0))
  return pl.pallas_call(
      _tc_body,
      grid=grid,
      in_specs=[row, row, full, full, full, full, full, full,
                vec, vec, vec, vec],
      out_specs=[row, row],
      out_shape=[jax.ShapeDtypeStruct((N, D), jnp.float32),
                 jax.ShapeDtypeStruct((N, D), jnp.float32)],
  )(x, agg, W_l1, W_r1, W_res1, W_l2, W_r2, W_res2,
    b_l1.reshape(1, D), b_res1.reshape(1, D),
    b_l2.reshape(1, D), b_res2.reshape(1, D))


def kernel(x, edge_index, W_l1, b_l1, W_r1, W_res1, b_res1,
           W_l2, b_l2, W_r2, W_res2, b_res2):
  src = edge_index[0]
  dst = edge_index[1]
  src3 = src.reshape(NS, STEPS, G)
  dst_c0 = jnp.where(dst < HALF, dst, TRASH)
  dst_c1 = jnp.where(dst >= HALF, dst - HALF, TRASH)
  dst4 = jnp.stack([dst_c0, dst_c1]).reshape(NC, NS, STEPS, G)
  xc = jnp.stack([x[:, :DH], x[:, DH:]])

  agg_pad = _segment_sum_sc(xc, src3, dst4)
  agg = jnp.concatenate(
      [jnp.concatenate([agg_pad[0, 0, :HALF], agg_pad[0, 1, :HALF]], axis=1),
       jnp.concatenate([agg_pad[1, 0, :HALF], agg_pad[1, 1, :HALF]], axis=1)],
      axis=0)

  loc, scale = _dense_tc(x, agg, W_l1, W_r1, W_res1, W_l2, W_r2, W_res2,
                         b_l1, b_res1, b_l2, b_res2)
  return (loc, scale)


# speedup vs baseline: 4.3329x; 1.0377x over previous
import jax
import jax.numpy as jnp
from jax import lax
from jax.experimental import pallas as pl
from jax.experimental.pallas import tpu as pltpu
from jax.experimental.pallas import tpu_sc as plsc

N = 10000
E = 160000
D = 256
DH = 128
RW = 0.001

NC = 2
NS = 16
HALF = N // NC
ROWS_PER_TILE = 320
PAD = NS * ROWS_PER_TILE
TRASH = HALF
EPT = E // NS
G = 80
STEPS = EPT // G
SB = 64


def _sc_body(xc_hbm, src_hbm, dst_hbm, out_hbm, src_v, dst_v, rows_v,
             stage_v, sem0, sem1, agg_sh):
  c = lax.axis_index("c")
  s = lax.axis_index("s")

  zero16 = jnp.zeros((16,), jnp.float32)

  def zrow(i, _):
    def zcol(j, _):
      stage_v[i, pl.ds(j * 16, 16)] = zero16
      return 0
    return lax.fori_loop(0, DH // 16, zcol, 0)

  lax.fori_loop(0, SB, zrow, 0)

  pltpu.sync_copy(src_hbm.at[s], src_v)
  pltpu.sync_copy(dst_hbm.at[c, s], dst_v)

  for p in range(2):
    def zs(q, _):
      pltpu.sync_copy(stage_v,
                      agg_sh.at[pl.ds(s * ROWS_PER_TILE + q * SB, SB)])
      return 0

    lax.fori_loop(0, ROWS_PER_TILE // SB, zs, 0)
    plsc.subcore_barrier()

    pltpu.async_copy(xc_hbm.at[p].at[src_v.at[0]], rows_v.at[0], sem0)

    def step(j, _):
      even = lax.rem(j, 2) == 0

      @pl.when((j + 1 < STEPS) & even)
      def _():
        pltpu.async_copy(xc_hbm.at[p].at[src_v.at[j + 1]], rows_v.at[1],
                         sem1)

      @pl.when((j + 1 < STEPS) & jnp.logical_not(even))
      def _():
        pltpu.async_copy(xc_hbm.at[p].at[src_v.at[j + 1]], rows_v.at[0],
                         sem0)

      @pl.when(even)
      def _():
        pltpu.make_async_copy(xc_hbm.at[p].at[src_v.at[j]], rows_v.at[0],
                              sem0).wait()
        pltpu.sync_copy(rows_v.at[0], agg_sh.at[dst_v.at[j]], add=True)

      @pl.when(jnp.logical_not(even))
      def _():
        pltpu.make_async_copy(xc_hbm.at[p].at[src_v.at[j]], rows_v.at[1],
                              sem1).wait()
        pltpu.sync_copy(rows_v.at[1], agg_sh.at[dst_v.at[j]], add=True)

      return 0

    lax.fori_loop(0, STEPS, step, 0)
    plsc.subcore_barrier()

    def co(q, _):
      pltpu.sync_copy(agg_sh.at[pl.ds(s * ROWS_PER_TILE + q * SB, SB)],
                      stage_v)
      pltpu.sync_copy(stage_v,
                      out_hbm.at[c, p, pl.ds(s * ROWS_PER_TILE + q * SB, SB)])
      return 0

    lax.fori_loop(0, ROWS_PER_TILE // SB, co, 0)
    if p == 0:
      lax.fori_loop(0, SB, zrow, 0)


@jax.jit
def _segment_sum_sc(xc, src3, dst4):
  mesh = plsc.VectorSubcoreMesh(core_axis_name="c", subcore_axis_name="s",
                                num_cores=NC, num_subcores=NS)
  f = pl.kernel(
      _sc_body,
      out_type=jax.ShapeDtypeStruct((NC, 2, PAD, DH), jnp.float32),
      mesh=mesh,
      scratch_types=[
          pltpu.VMEM((STEPS, G), jnp.int32),
          pltpu.VMEM((STEPS, G), jnp.int32),
          pltpu.VMEM((2, G, DH), jnp.float32),
          pltpu.VMEM((SB, DH), jnp.float32),
          pltpu.SemaphoreType.DMA,
          pltpu.SemaphoreType.DMA,
          pltpu.VMEM_SHARED((PAD, DH), jnp.float32),
      ],
  )
  return f(xc, src3, dst4)


def _tc_body(x_ref, agg_ref, wl1, wr1, wres1, wl2, wr2, wres2,
             bl1, bres1, bl2, bres2, loc_ref, scale_ref):
  ap = agg_ref[...]
  a1 = RW * wl1[...]
  b1 = RW * wr1[...] + (1.0 - RW) * wres1[...]
  a2 = RW * wl2[...]
  b2 = RW * wr2[...] + (1.0 - RW) * wres2[...]
  c1 = RW * bl1[...] + (1.0 - RW) * bres1[...]
  c2 = RW * bl2[...] + (1.0 - RW) * bres2[...]
  hs = jnp.concatenate([ap[0, 0], ap[0, 1], x_ref[...]], axis=1)
  wcat = jnp.concatenate(
      [jnp.concatenate([a1, b1], axis=1),
       jnp.concatenate([a2, b2], axis=1)], axis=0)
  hall = lax.dot_general(hs, wcat, (((1,), (1,)), ((), ())),
                         preferred_element_type=jnp.float32)
  h1 = hall[:, :D] + c1
  h2 = hall[:, D:] + c2
  loc_ref[...] = jnp.clip(h1, -100.0, 100.0)
  scale_ref[...] = jnp.minimum(jax.nn.softplus(h2) + 0.001, 100.0)


@jax.jit
def _dense_tc(x, agg, W_l1, W_r1, W_res1, W_l2, W_r2, W_res2,
              b_l1, b_res1, b_l2, b_res2):
  bm = 1000
  grid = (N // bm,)
  nb = HALF // bm
  row = pl.BlockSpec((bm, D), lambda i: (i, 0))
  rowp = pl.BlockSpec((1, 2, bm, DH), lambda i: (i // nb, 0, i % nb, 0))
  full = pl.BlockSpec((D, D), lambda i: (0, 0))
  vec = pl.BlockSpec((1, D), lambda i: (0, 0))
  return pl.pallas_call(
      _tc_body,
      grid=grid,
      in_specs=[row, rowp, full, full, full, full, full, full,
                vec, vec, vec, vec],
      out_specs=[row, row],
      out_shape=[jax.ShapeDtypeStruct((N, D), jnp.float32),
                 jax.ShapeDtypeStruct((N, D), jnp.float32)],
  )(x, agg, W_l1, W_r1, W_res1, W_l2, W_r2, W_res2,
    b_l1.reshape(1, D), b_res1.reshape(1, D),
    b_l2.reshape(1, D), b_res2.reshape(1, D))


def kernel(x, edge_index, W_l1, b_l1, W_r1, W_res1, b_res1,
           W_l2, b_l2, W_r2, W_res2, b_res2):
  src = edge_index[0]
  dst = edge_index[1]
  src3 = src.reshape(NS, STEPS, G)
  dst_c0 = jnp.where(dst < HALF, dst, TRASH)
  dst_c1 = jnp.where(dst >= HALF, dst - HALF, TRASH)
  dst4 = jnp.stack([dst_c0, dst_c1]).reshape(NC, NS, STEPS, G)
  xc = jnp.stack([x[:, :DH], x[:, DH:]])

  agg_pad = _segment_sum_sc(xc, src3, dst4)

  loc, scale = _dense_tc(x, agg_pad, W_l1, W_r1, W_res1, W_l2, W_r2,
                         W_res2, b_l1, b_res1, b_l2, b_res2)
  return (loc, scale)
